# baseline (speedup 1.0000x reference)
import jax
import jax.numpy as jnp
from jax import lax
from jax.experimental import pallas as pl

N_OBJ = 128
T = 100
D = 64
ROWS = 128


def _concat_body(ids_ref, s_ref, tab_ref, o_ref):
    ids = ids_ref[...]
    cols = lax.broadcasted_iota(jnp.int32, (ROWS, N_OBJ), 1)
    onehot = (ids == cols).astype(jnp.float32)
    emb = jnp.dot(onehot, tab_ref[...],
                  preferred_element_type=jnp.float32)
    embb = jnp.broadcast_to(emb[:, None, :], (ROWS, T, D))
    o_ref[...] = jnp.concatenate([s_ref[...], embb], axis=-1)


def kernel(states, table, obj_ids):
    B, N, t, d = states.shape
    flat = states.reshape(B * N, t, d)
    ids2d = obj_ids.reshape(N, 1)
    grid = (B * N) // ROWS
    out = pl.pallas_call(
        _concat_body,
        grid=(grid,),
        in_specs=[
            pl.BlockSpec((ROWS, 1), lambda g: (g % (N_OBJ // ROWS), 0)),
            pl.BlockSpec((ROWS, t, d), lambda g: (g, 0, 0)),
            pl.BlockSpec((N_OBJ, d), lambda g: (0, 0)),
        ],
        out_specs=pl.BlockSpec((ROWS, t, 2 * d), lambda g: (g, 0, 0)),
        out_shape=jax.ShapeDtypeStruct((B * N, t, 2 * d), jnp.float32),
    )(ids2d, flat, table)
    return out.reshape(B, N, t, 2 * d)
